# Initial kernel scaffold; baseline (speedup 1.0000x reference)
import jax
import jax.numpy as jnp
from jax.experimental import pallas as pl


def kernel(x, union_features, box_info, obj_labels, rel_labels, rel_pair_idx, params):
    raise NotImplementedError("write your pallas kernel here")


# speedup vs baseline: 2.0825x; 2.0825x over previous
import functools

import jax
import jax.numpy as jnp
from jax import lax
from jax.experimental import pallas as pl
from jax.experimental.pallas import tpu as pltpu
from jax.experimental.pallas import tpu_sc as plsc

N = 512
E = 8192
RD = 64
OD = 256
PD = 512
NOC = 151

NS = 16
EPT = E // NS
CH = 4
CSZ = EPT // CH
RPT = N // NS


def _prep_body(x_ref, union_ref, box_ref, obj_ref, rel_ref,
               posW1_ref, posb1_ref, posg_ref, posbt_ref, posW2_ref, posb2_ref,
               Wobj_ref, bobj_ref, Wlbla_ref, Wlblb_ref, blbl_ref,
               Wrel_ref, brel_ref, Wrlbl_ref, brlbl_ref,
               imUs_ref, imUd_ref, imWh_ref, imbe_ref, imbh_ref, imWe_ref,
               sgUs_ref, sgUd_ref, sgWh_ref, sgbe_ref, sgbh_ref, sgWe_ref,
               s2_ref, d2_ref, hWh2_ref, efW2_ref):
    f32 = jnp.float32
    dot = functools.partial(jnp.dot, preferred_element_type=f32)

    a = dot(box_ref[...], posW1_ref[...]) + posb1_ref[...]
    mu = jnp.mean(a, axis=0, keepdims=True)
    var = jnp.mean((a - mu) ** 2, axis=0, keepdims=True)
    bn = posg_ref[...] * (a - mu) / jnp.sqrt(var + 1e-5) + posbt_ref[...]
    pos = jnp.maximum(dot(bn, posW2_ref[...]) + posb2_ref[...], 0.0)

    h_im = dot(x_ref[...], Wobj_ref[...]) + bobj_ref[...]
    h_sg = dot(obj_ref[...], Wlbla_ref[...]) + dot(pos, Wlblb_ref[...]) + blbl_ref[...]

    s2_ref[0] = dot(h_im, imUs_ref[...]) + imbe_ref[...]
    s2_ref[1] = dot(h_sg, sgUs_ref[...]) + sgbe_ref[...]
    d2_ref[0] = dot(h_im, imUd_ref[...])
    d2_ref[1] = dot(h_sg, sgUd_ref[...])
    hWh2_ref[0] = dot(h_im, imWh_ref[...]) + imbh_ref[...]
    hWh2_ref[1] = dot(h_sg, sgWh_ref[...]) + sgbh_ref[...]

    dotl = functools.partial(jnp.dot, preferred_element_type=f32)
    Wf_im = dot(Wrel_ref[...], imWe_ref[...])
    bf_im = dot(brel_ref[...], imWe_ref[...])
    efW2_ref[0] = dotl(union_ref[...], Wf_im) + bf_im
    Wf_sg = dot(Wrlbl_ref[...], sgWe_ref[...])
    bf_sg = dot(brlbl_ref[...], sgWe_ref[...])
    efW2_ref[1] = dotl(rel_ref[...], Wf_sg) + bf_sg


def _prep_call(x, union, box, obj, rel, weights):
    f32 = jnp.float32
    out_shape = [
        jax.ShapeDtypeStruct((2, N, RD), f32),
        jax.ShapeDtypeStruct((2, N, RD), f32),
        jax.ShapeDtypeStruct((2, N, OD), f32),
        jax.ShapeDtypeStruct((2, E, RD), f32),
    ]
    return pl.pallas_call(_prep_body, out_shape=out_shape)(
        x, union, box, obj, rel, *weights)


KPT = (N * N) // NS
KSH = 14
EC = E // CSZ


def _sc_body(i2d_hbm, j2d_hbm, s2f_hbm, d2f_hbm, efw2f_hbm,
             corr_out,
             acc_sh, corr_sh, own_tab,
             iv, jv, kv, ev, ov,
             map_loc, keysf, ownf, idx64,
             srow, drow, accg, tpay, zbuf, zbuf2):
    c = lax.axis_index("c")
    s = lax.axis_index("s")
    base = s * EPT

    zvec = jnp.zeros((16,), jnp.float32)
    for cc in range(RD // 16):
        def zb(r, _, cc=cc):
            zbuf[r, pl.ds(cc * 16, 16)] = zvec
            return 0
        lax.fori_loop(0, RPT, zb, 0)
    for cc in range(2 * RD // 16):
        def zb2(r, _, cc=cc):
            zbuf2[r, pl.ds(cc * 16, 16)] = zvec
            return 0
        lax.fori_loop(0, RPT, zb2, 0)

    lane = lax.iota(jnp.int32, 16)
    zivec = jnp.zeros((16,), jnp.int32)

    pltpu.sync_copy(i2d_hbm.at[pl.ds(s * CH, CH)], iv)
    pltpu.sync_copy(j2d_hbm.at[pl.ds(s * CH, CH)], jv)
    for k in range(CH):
        for v in range(CSZ // 16):
            sl = pl.ds(v * 16, 16)
            kv[k, sl] = c * N + iv[k, sl]
            ev[k, sl] = c * N + jv[k, sl]
            ov[k, sl] = zivec
    pltpu.sync_copy(ov, own_tab.at[pl.ds(s * CH, CH)])

    def zacc(q, _):
        pltpu.sync_copy(zbuf, acc_sh.at[pl.ds(base + q * RPT, RPT)])
        return 0
    lax.fori_loop(0, EPT // RPT, zacc, 0)
    pltpu.sync_copy(zbuf2, corr_sh.at[pl.ds(s * RPT, RPT)])

    def kb(q, _):
        pltpu.sync_copy(i2d_hbm.at[pl.ds(q * CH, CH)], ov)
        pltpu.sync_copy(j2d_hbm.at[pl.ds(q * CH, CH)], jv)
        for k in range(CH):
            for v in range(CSZ // 16):
                sl = pl.ds(v * 16, 16)
                keysf[q * CH + k, sl] = ov[k, sl] * N + jv[k, sl]
        return 0
    lax.fori_loop(0, EC // CH, kb, 0)

    plsc.subcore_barrier()

    def sc1(t, _):
        r = t // 8
        vv = lax.rem(t, 8)
        sl = pl.ds(vv * 16, 16)
        kvec = keysf[r, sl]
        m = lax.shift_right_logical(kvec, KSH) == s
        loc = lax.bitwise_and(kvec, KPT - 1)
        plsc.store_scatter(map_loc, [loc], t * 16 + lane, mask=m)
        return 0
    lax.fori_loop(0, E // 16, sc1, 0)

    def sc2(t, _):
        r = t // 8
        vv = lax.rem(t, 8)
        sl = pl.ds(vv * 16, 16)
        kvec = keysf[r, sl]
        m = lax.shift_right_logical(kvec, KSH) == s
        loc = lax.bitwise_and(kvec, KPT - 1)
        g = plsc.load_gather(map_loc, [loc], mask=m)
        ownf[r, sl] = jnp.where(m, g, 0)
        return 0
    lax.fori_loop(0, E // 16, sc2, 0)

    for w in range(EC // 16):
        idx64[0, pl.ds(w * 16, 16)] = w * 16 + lane
    pltpu.sync_copy(ownf, own_tab.at[idx64.at[0]], add=True)

    plsc.subcore_barrier()

    pltpu.sync_copy(own_tab.at[pl.ds(s * CH, CH)], ov)
    for k in range(CH):
        off = base + k * CSZ
        pltpu.sync_copy(efw2f_hbm.at[pl.ds(c * E + off, CSZ)], srow)
        pltpu.sync_copy(srow, acc_sh.at[ov.at[k]], add=True)

    plsc.subcore_barrier()

    for k in range(CH):
        off = base + k * CSZ
        pltpu.sync_copy(acc_sh.at[pl.ds(off, CSZ)], accg)
        pltpu.sync_copy(s2f_hbm.at[kv.at[k]], srow)
        pltpu.sync_copy(d2f_hbm.at[ev.at[k]], drow)

        def rb(r, _):
            for cc in range(RD // 16):
                sl = pl.ds(cc * 16, 16)
                b16 = srow[r, sl] + drow[r, sl]
                tpay[r, sl] = (jnp.maximum(b16 + accg[r, sl], 0.0)
                               - jnp.maximum(b16, 0.0))
                tpay[r, pl.ds(RD + cc * 16, 16)] = zvec
            return 0
        lax.fori_loop(0, CSZ, rb, 0)
        pltpu.sync_copy(tpay, corr_sh.at[iv.at[k]], add=True)

    plsc.subcore_barrier()

    pltpu.sync_copy(corr_sh.at[pl.ds(s * RPT, RPT)],
                    corr_out.at[pl.ds(c * N + s * RPT, RPT)])


def _sc_call(i_arr, j_arr, s2, d2, efw2):
    f32 = jnp.float32
    i32 = jnp.int32
    mesh = plsc.VectorSubcoreMesh(core_axis_name="c", subcore_axis_name="s",
                                  num_cores=2, num_subcores=NS)
    fn = pl.kernel(
        _sc_body,
        out_type=jax.ShapeDtypeStruct((2 * N, 2 * RD), f32),
        mesh=mesh,
        compiler_params=pltpu.CompilerParams(use_tc_tiling_on_sc=False, needs_layout_passes=False),
        scratch_types=[
            pltpu.VMEM_SHARED((E, RD), f32),
            pltpu.VMEM_SHARED((N, 2 * RD), f32),
            pltpu.VMEM_SHARED((EC, CSZ), i32),
            pltpu.VMEM((CH, CSZ), i32),
            pltpu.VMEM((CH, CSZ), i32),
            pltpu.VMEM((CH, CSZ), i32),
            pltpu.VMEM((CH, CSZ), i32),
            pltpu.VMEM((CH, CSZ), i32),
            pltpu.VMEM((KPT,), i32),
            pltpu.VMEM((EC, CSZ), i32),
            pltpu.VMEM((EC, CSZ), i32),
            pltpu.VMEM((1, EC), i32),
            pltpu.VMEM((CSZ, RD), f32),
            pltpu.VMEM((CSZ, RD), f32),
            pltpu.VMEM((CSZ, RD), f32),
            pltpu.VMEM((CSZ, 2 * RD), f32),
            pltpu.VMEM((RPT, RD), f32),
            pltpu.VMEM((RPT, 2 * RD), f32),
        ],
    )
    corr_flat = fn(i_arr.reshape(EC, CSZ), j_arr.reshape(EC, CSZ),
                   s2.reshape(2 * N, RD), d2.reshape(2 * N, RD),
                   efw2.reshape(2 * E, RD))
    return corr_flat.reshape(2, N, 2 * RD)


def _finale_body(s2_ref, d2_ref, corr_ref, hWh2_ref,
                 Pe2_ref, Wgnt2_ref, Wpnt2_ref, Wget2_ref, Wpet2_ref,
                 enW1_ref, enb1_ref, enW2t_ref, enb2_ref,
                 out_ref):
    f32 = jnp.float32
    dot = functools.partial(jnp.dot, preferred_element_type=f32)

    S = jnp.concatenate([s2_ref[0], s2_ref[1]], axis=1)

    IB = 128
    JB = 8
    rows = []
    for ib in range(N // IB):
        Sb = S[ib * IB:(ib + 1) * IB, :]

        def jb_body(jb, acc, Sb=Sb):
            Dblk = jnp.concatenate(
                [d2_ref[0, pl.ds(jb * JB, JB), :],
                 d2_ref[1, pl.ds(jb * JB, JB), :]], axis=1)
            for t in range(JB):
                acc = acc + jnp.maximum(Sb + Dblk[t:t + 1, :], 0.0)
            return acc
        rows.append(lax.fori_loop(0, N // JB, jb_body,
                                  jnp.zeros((IB, 2 * RD), f32)))
    DROW = jnp.concatenate(rows, axis=0)

    pooled = []
    for g in range(2):
        ROW = DROW[:, g * RD:(g + 1) * RD] + corr_ref[g]
        h2 = jnp.maximum(hWh2_ref[g] + dot(ROW, Pe2_ref[g]), 0.0)
        gn = jax.nn.sigmoid(jnp.sum(h2 * Wgnt2_ref[g], axis=1, keepdims=True))
        pn = dot(jnp.sum(gn * h2, axis=0, keepdims=True), Wpnt2_ref[g])
        ge = jax.nn.sigmoid(jnp.sum(ROW * Wget2_ref[g], axis=1, keepdims=True))
        pe = dot(jnp.sum(ge * ROW, axis=0, keepdims=True), Wpet2_ref[g])
        pooled.append(pn + pe)

    hcat = jnp.concatenate(pooled, axis=1)
    e1 = jnp.maximum(dot(hcat, enW1_ref[...]) + enb1_ref[...], 0.0)
    out_ref[...] = dot(e1, enW2t_ref[...]) + enb2_ref[...]


def _finale_call(s2, d2, corr2, hWh2, heads):
    return pl.pallas_call(
        _finale_body,
        out_shape=jax.ShapeDtypeStruct((1, 1), jnp.float32),
    )(s2, d2, corr2, hWh2, *heads)


def kernel(x, union_features, box_info, obj_labels, rel_labels, rel_pair_idx,
           params):
    p = params
    r2 = lambda v: v.reshape(1, -1)

    weights = (
        p['pos_W1'], r2(p['pos_b1']), r2(p['pos_g']), r2(p['pos_bt']),
        p['pos_W2'], r2(p['pos_b2']),
        p['W_obj_emb'], r2(p['b_obj_emb']),
        p['W_obj_lbl'][:NOC], p['W_obj_lbl'][NOC:], r2(p['b_obj_lbl']),
        p['W_rel_emb'], r2(p['b_rel_emb']), p['W_rel_lbl'], r2(p['b_rel_lbl']),
        p['im_Us'], p['im_Ud'], p['im_Wh'], r2(p['im_be']), r2(p['im_bh']), p['im_We'],
        p['sg_Us'], p['sg_Ud'], p['sg_Wh'], r2(p['sg_be']), r2(p['sg_bh']), p['sg_We'],
    )
    s2, d2, hWh2, efW2 = _prep_call(x, union_features, box_info, obj_labels,
                                    rel_labels, weights)

    i_arr = rel_pair_idx[:, 0]
    j_arr = rel_pair_idx[:, 1]
    corr2 = _sc_call(i_arr, j_arr, s2, d2, efW2)[:, :, :RD]

    heads = (
        jnp.stack([p['im_Pe'], p['sg_Pe']]),
        jnp.stack([p['imp_Wgn'].T, p['sgp_Wgn'].T]),
        jnp.stack([p['imp_Wpn'], p['sgp_Wpn']]),
        jnp.stack([p['imp_Wge'].T, p['sgp_Wge'].T]),
        jnp.stack([p['imp_Wpe'], p['sgp_Wpe']]),
        p['en_W1'], r2(p['en_b1']), p['en_W2'], r2(p['en_b2']),
    )
    return _finale_call(s2, d2, corr2, hWh2, heads)


# speedup vs baseline: 2.3087x; 1.1086x over previous
import functools

import jax
import jax.numpy as jnp
from jax import lax
from jax.experimental import pallas as pl
from jax.experimental.pallas import tpu as pltpu
from jax.experimental.pallas import tpu_sc as plsc

N = 512
E = 8192
RD = 64
OD = 256
PD = 512
NOC = 151

NS = 16
EPT = E // NS
CH = 4
CSZ = EPT // CH
RPT = N // NS


def _prep_body(x_ref, union_ref, box_ref, obj_ref, rel_ref,
               posW1_ref, posb1_ref, posg_ref, posbt_ref, posW2_ref, posb2_ref,
               Wobj_ref, bobj_ref, Wlbla_ref, Wlblb_ref, blbl_ref,
               Wrel_ref, brel_ref, Wrlbl_ref, brlbl_ref,
               imUs_ref, imUd_ref, imWh_ref, imbe_ref, imbh_ref, imWe_ref,
               sgUs_ref, sgUd_ref, sgWh_ref, sgbe_ref, sgbh_ref, sgWe_ref,
               s2_ref, d2_ref, hWh2_ref, efW2_ref):
    f32 = jnp.float32
    dot = functools.partial(jnp.dot, preferred_element_type=f32)

    a = dot(box_ref[...], posW1_ref[...]) + posb1_ref[...]
    mu = jnp.mean(a, axis=0, keepdims=True)
    var = jnp.mean((a - mu) ** 2, axis=0, keepdims=True)
    bn = posg_ref[...] * (a - mu) / jnp.sqrt(var + 1e-5) + posbt_ref[...]
    pos = jnp.maximum(dot(bn, posW2_ref[...]) + posb2_ref[...], 0.0)

    h_im = dot(x_ref[...], Wobj_ref[...]) + bobj_ref[...]
    h_sg = dot(obj_ref[...], Wlbla_ref[...]) + dot(pos, Wlblb_ref[...]) + blbl_ref[...]

    s2_ref[0] = dot(h_im, imUs_ref[...]) + imbe_ref[...]
    s2_ref[1] = dot(h_sg, sgUs_ref[...]) + sgbe_ref[...]
    d2_ref[0] = dot(h_im, imUd_ref[...])
    d2_ref[1] = dot(h_sg, sgUd_ref[...])
    hWh2_ref[0] = dot(h_im, imWh_ref[...]) + imbh_ref[...]
    hWh2_ref[1] = dot(h_sg, sgWh_ref[...]) + sgbh_ref[...]

    dotl = functools.partial(jnp.dot, preferred_element_type=f32)
    Wf_im = dot(Wrel_ref[...], imWe_ref[...])
    bf_im = dot(brel_ref[...], imWe_ref[...])
    efW2_ref[0] = dotl(union_ref[...], Wf_im) + bf_im
    Wf_sg = dot(Wrlbl_ref[...], sgWe_ref[...])
    bf_sg = dot(brlbl_ref[...], sgWe_ref[...])
    efW2_ref[1] = dotl(rel_ref[...], Wf_sg) + bf_sg


def _prep_call(x, union, box, obj, rel, weights):
    f32 = jnp.float32
    out_shape = [
        jax.ShapeDtypeStruct((2, N, RD), f32),
        jax.ShapeDtypeStruct((2, N, RD), f32),
        jax.ShapeDtypeStruct((2, N, OD), f32),
        jax.ShapeDtypeStruct((2, E, RD), f32),
    ]
    return pl.pallas_call(_prep_body, out_shape=out_shape)(
        x, union, box, obj, rel, *weights)


KPT = (N * N) // NS
KSH = 14
EC = E // CSZ


def _sc_body(i2d_hbm, j2d_hbm, s2f_hbm, d2f_hbm, efw2f_hbm,
             corr_out,
             acc_sh, corr_sh, own_tab,
             ibuf, jbuf, kv, ev, ov,
             map_loc, ownf, idx64,
             srow, drow, accg, tpay, zbuf, zbuf2):
    c = lax.axis_index("c")
    s = lax.axis_index("s")
    base = s * EPT

    zvec = jnp.zeros((16,), jnp.float32)
    for cc in range(RD // 16):
        def zb(r, _, cc=cc):
            zbuf[r, pl.ds(cc * 16, 16)] = zvec
            return 0
        lax.fori_loop(0, RPT, zb, 0)
    for cc in range(2 * RD // 16):
        def zb2(r, _, cc=cc):
            zbuf2[r, pl.ds(cc * 16, 16)] = zvec
            return 0
        lax.fori_loop(0, RPT, zb2, 0)

    lane = lax.iota(jnp.int32, 16)
    zivec = jnp.zeros((16,), jnp.int32)

    pltpu.sync_copy(i2d_hbm, ibuf)
    pltpu.sync_copy(j2d_hbm, jbuf)
    for k in range(CH):
        for v in range(CSZ // 16):
            sl = pl.ds(v * 16, 16)
            kv[k, sl] = c * N + ibuf[s * CH + k, sl]
            ev[k, sl] = c * N + jbuf[s * CH + k, sl]
            ov[k, sl] = zivec
    pltpu.sync_copy(ov, own_tab.at[pl.ds(s * CH, CH)])

    def zacc(q, _):
        pltpu.sync_copy(zbuf, acc_sh.at[pl.ds(base + q * RPT, RPT)])
        return 0
    lax.fori_loop(0, EPT // RPT, zacc, 0)
    pltpu.sync_copy(zbuf2, corr_sh.at[pl.ds(s * RPT, RPT)])

    plsc.subcore_barrier()

    def sc1(t, _):
        r = t // 8
        vv = lax.rem(t, 8)
        sl = pl.ds(vv * 16, 16)
        kvec = ibuf[r, sl] * N + jbuf[r, sl]
        m = lax.shift_right_logical(kvec, KSH) == s
        loc = lax.bitwise_and(kvec, KPT - 1)
        plsc.store_scatter(map_loc, [loc], t * 16 + lane, mask=m)
        return 0
    lax.fori_loop(0, E // 16, sc1, 0)

    def sc2(t, _):
        r = t // 8
        vv = lax.rem(t, 8)
        sl = pl.ds(vv * 16, 16)
        kvec = ibuf[r, sl] * N + jbuf[r, sl]
        m = lax.shift_right_logical(kvec, KSH) == s
        loc = lax.bitwise_and(kvec, KPT - 1)
        g = plsc.load_gather(map_loc, [loc], mask=m)
        ownf[r, sl] = jnp.where(m, g, 0)
        return 0
    lax.fori_loop(0, E // 16, sc2, 0)

    for w in range(EC // 16):
        idx64[0, pl.ds(w * 16, 16)] = w * 16 + lane
    pltpu.sync_copy(ownf, own_tab.at[idx64.at[0]], add=True)

    plsc.subcore_barrier()

    pltpu.sync_copy(own_tab.at[pl.ds(s * CH, CH)], ov)
    for k in range(CH):
        off = base + k * CSZ
        pltpu.sync_copy(efw2f_hbm.at[pl.ds(c * E + off, CSZ)], srow)
        pltpu.sync_copy(srow, acc_sh.at[ov.at[k]], add=True)

    plsc.subcore_barrier()

    for k in range(CH):
        off = base + k * CSZ
        pltpu.sync_copy(acc_sh.at[pl.ds(off, CSZ)], accg)
        pltpu.sync_copy(s2f_hbm.at[kv.at[k]], srow)
        pltpu.sync_copy(d2f_hbm.at[ev.at[k]], drow)

        def rb(r, _):
            for cc in range(RD // 16):
                sl = pl.ds(cc * 16, 16)
                b16 = srow[r, sl] + drow[r, sl]
                tpay[r, sl] = (jnp.maximum(b16 + accg[r, sl], 0.0)
                               - jnp.maximum(b16, 0.0))
                tpay[r, pl.ds(RD + cc * 16, 16)] = zvec
            return 0
        lax.fori_loop(0, CSZ, rb, 0)
        pltpu.sync_copy(tpay, corr_sh.at[ibuf.at[s * CH + k]], add=True)

    plsc.subcore_barrier()

    pltpu.sync_copy(corr_sh.at[pl.ds(s * RPT, RPT)],
                    corr_out.at[pl.ds(c * N + s * RPT, RPT)])


def _sc_call(i_arr, j_arr, s2, d2, efw2):
    f32 = jnp.float32
    i32 = jnp.int32
    mesh = plsc.VectorSubcoreMesh(core_axis_name="c", subcore_axis_name="s",
                                  num_cores=2, num_subcores=NS)
    fn = pl.kernel(
        _sc_body,
        out_type=jax.ShapeDtypeStruct((2 * N, 2 * RD), f32),
        mesh=mesh,
        compiler_params=pltpu.CompilerParams(use_tc_tiling_on_sc=False, needs_layout_passes=False),
        scratch_types=[
            pltpu.VMEM_SHARED((E, RD), f32),
            pltpu.VMEM_SHARED((N, 2 * RD), f32),
            pltpu.VMEM_SHARED((EC, CSZ), i32),
            pltpu.VMEM((EC, CSZ), i32),
            pltpu.VMEM((EC, CSZ), i32),
            pltpu.VMEM((CH, CSZ), i32),
            pltpu.VMEM((CH, CSZ), i32),
            pltpu.VMEM((CH, CSZ), i32),
            pltpu.VMEM((KPT,), i32),
            pltpu.VMEM((EC, CSZ), i32),
            pltpu.VMEM((1, EC), i32),
            pltpu.VMEM((CSZ, RD), f32),
            pltpu.VMEM((CSZ, RD), f32),
            pltpu.VMEM((CSZ, RD), f32),
            pltpu.VMEM((CSZ, 2 * RD), f32),
            pltpu.VMEM((RPT, RD), f32),
            pltpu.VMEM((RPT, 2 * RD), f32),
        ],
    )
    corr_flat = fn(i_arr.reshape(EC, CSZ), j_arr.reshape(EC, CSZ),
                   s2.reshape(2 * N, RD), d2.reshape(2 * N, RD),
                   efw2.reshape(2 * E, RD))
    return corr_flat.reshape(2, N, 2 * RD)


def _finale_body(s2_ref, d2_ref, corr_ref, hWh2_ref,
                 Pe2_ref, Wgnt2_ref, Wpnt2_ref, Wget2_ref, Wpet2_ref,
                 enW1_ref, enb1_ref, enW2t_ref, enb2_ref,
                 out_ref):
    f32 = jnp.float32
    dot = functools.partial(jnp.dot, preferred_element_type=f32)

    S = jnp.concatenate([s2_ref[0], s2_ref[1]], axis=1)

    IB = 128
    JB = 8
    rows = []
    for ib in range(N // IB):
        Sb = S[ib * IB:(ib + 1) * IB, :]

        def jb_body(jb, acc, Sb=Sb):
            Dblk = jnp.concatenate(
                [d2_ref[0, pl.ds(jb * JB, JB), :],
                 d2_ref[1, pl.ds(jb * JB, JB), :]], axis=1)
            for t in range(JB):
                acc = acc + jnp.maximum(Sb + Dblk[t:t + 1, :], 0.0)
            return acc
        rows.append(lax.fori_loop(0, N // JB, jb_body,
                                  jnp.zeros((IB, 2 * RD), f32)))
    DROW = jnp.concatenate(rows, axis=0)

    pooled = []
    for g in range(2):
        ROW = DROW[:, g * RD:(g + 1) * RD] + corr_ref[g]
        h2 = jnp.maximum(hWh2_ref[g] + dot(ROW, Pe2_ref[g]), 0.0)
        gn = jax.nn.sigmoid(jnp.sum(h2 * Wgnt2_ref[g], axis=1, keepdims=True))
        pn = dot(jnp.sum(gn * h2, axis=0, keepdims=True), Wpnt2_ref[g])
        ge = jax.nn.sigmoid(jnp.sum(ROW * Wget2_ref[g], axis=1, keepdims=True))
        pe = dot(jnp.sum(ge * ROW, axis=0, keepdims=True), Wpet2_ref[g])
        pooled.append(pn + pe)

    hcat = jnp.concatenate(pooled, axis=1)
    e1 = jnp.maximum(dot(hcat, enW1_ref[...]) + enb1_ref[...], 0.0)
    out_ref[...] = dot(e1, enW2t_ref[...]) + enb2_ref[...]


def _finale_call(s2, d2, corr2, hWh2, heads):
    return pl.pallas_call(
        _finale_body,
        out_shape=jax.ShapeDtypeStruct((1, 1), jnp.float32),
    )(s2, d2, corr2, hWh2, *heads)


def kernel(x, union_features, box_info, obj_labels, rel_labels, rel_pair_idx,
           params):
    p = params
    r2 = lambda v: v.reshape(1, -1)

    weights = (
        p['pos_W1'], r2(p['pos_b1']), r2(p['pos_g']), r2(p['pos_bt']),
        p['pos_W2'], r2(p['pos_b2']),
        p['W_obj_emb'], r2(p['b_obj_emb']),
        p['W_obj_lbl'][:NOC], p['W_obj_lbl'][NOC:], r2(p['b_obj_lbl']),
        p['W_rel_emb'], r2(p['b_rel_emb']), p['W_rel_lbl'], r2(p['b_rel_lbl']),
        p['im_Us'], p['im_Ud'], p['im_Wh'], r2(p['im_be']), r2(p['im_bh']), p['im_We'],
        p['sg_Us'], p['sg_Ud'], p['sg_Wh'], r2(p['sg_be']), r2(p['sg_bh']), p['sg_We'],
    )
    s2, d2, hWh2, efW2 = _prep_call(x, union_features, box_info, obj_labels,
                                    rel_labels, weights)

    i_arr = rel_pair_idx[:, 0]
    j_arr = rel_pair_idx[:, 1]
    corr2 = _sc_call(i_arr, j_arr, s2, d2, efW2)[:, :, :RD]

    heads = (
        jnp.stack([p['im_Pe'], p['sg_Pe']]),
        jnp.stack([p['imp_Wgn'].T, p['sgp_Wgn'].T]),
        jnp.stack([p['imp_Wpn'], p['sgp_Wpn']]),
        jnp.stack([p['imp_Wge'].T, p['sgp_Wge'].T]),
        jnp.stack([p['imp_Wpe'], p['sgp_Wpe']]),
        p['en_W1'], r2(p['en_b1']), p['en_W2'], r2(p['en_b2']),
    )
    return _finale_call(s2, d2, corr2, hWh2, heads)


# speedup vs baseline: 2.3848x; 1.0330x over previous
import functools

import jax
import jax.numpy as jnp
from jax import lax
from jax.experimental import pallas as pl
from jax.experimental.pallas import tpu as pltpu
from jax.experimental.pallas import tpu_sc as plsc

N = 512
E = 8192
RD = 64
OD = 256
PD = 512
NOC = 151

NS = 16
EPT = E // NS
CH = 4
CSZ = EPT // CH
RPT = N // NS


def _prep_body(x_ref, union_ref, box_ref, obj_ref, rel_ref,
               posW1_ref, posb1_ref, posg_ref, posbt_ref, posW2_ref, posb2_ref,
               Wobj_ref, bobj_ref, Wlbla_ref, Wlblb_ref, blbl_ref,
               Wrel_ref, brel_ref, Wrlbl_ref, brlbl_ref,
               imUs_ref, imUd_ref, imWh_ref, imbe_ref, imbh_ref, imWe_ref,
               sgUs_ref, sgUd_ref, sgWh_ref, sgbe_ref, sgbh_ref, sgWe_ref,
               s2_ref, d2_ref, hWh2_ref, efW2_ref):
    f32 = jnp.float32
    dot = functools.partial(jnp.dot, preferred_element_type=f32)

    a = dot(box_ref[...], posW1_ref[...]) + posb1_ref[...]
    mu = jnp.mean(a, axis=0, keepdims=True)
    var = jnp.mean((a - mu) ** 2, axis=0, keepdims=True)
    bn = posg_ref[...] * (a - mu) / jnp.sqrt(var + 1e-5) + posbt_ref[...]
    pos = jnp.maximum(dot(bn, posW2_ref[...]) + posb2_ref[...], 0.0)

    h_im = dot(x_ref[...], Wobj_ref[...]) + bobj_ref[...]
    h_sg = dot(obj_ref[...], Wlbla_ref[...]) + dot(pos, Wlblb_ref[...]) + blbl_ref[...]

    s2_ref[0] = dot(h_im, imUs_ref[...]) + imbe_ref[...]
    s2_ref[1] = dot(h_sg, sgUs_ref[...]) + sgbe_ref[...]
    d2_ref[0] = dot(h_im, imUd_ref[...])
    d2_ref[1] = dot(h_sg, sgUd_ref[...])
    hWh2_ref[0] = dot(h_im, imWh_ref[...]) + imbh_ref[...]
    hWh2_ref[1] = dot(h_sg, sgWh_ref[...]) + sgbh_ref[...]

    dotl = functools.partial(jnp.dot, preferred_element_type=f32)
    Wf_im = dot(Wrel_ref[...], imWe_ref[...])
    bf_im = dot(brel_ref[...], imWe_ref[...])
    efW2_ref[0] = dotl(union_ref[...], Wf_im) + bf_im
    Wf_sg = dot(Wrlbl_ref[...], sgWe_ref[...])
    bf_sg = dot(brlbl_ref[...], sgWe_ref[...])
    efW2_ref[1] = dotl(rel_ref[...], Wf_sg) + bf_sg


def _prep_call(x, union, box, obj, rel, weights):
    f32 = jnp.float32
    out_shape = [
        jax.ShapeDtypeStruct((2, N, RD), f32),
        jax.ShapeDtypeStruct((2, N, RD), f32),
        jax.ShapeDtypeStruct((2, N, OD), f32),
        jax.ShapeDtypeStruct((2, E, RD), f32),
    ]
    return pl.pallas_call(_prep_body, out_shape=out_shape)(
        x, union, box, obj, rel, *weights)


KPT = (N * N) // NS
KSH = 14
EC = E // CSZ


def _sc_body(i2d_hbm, j2d_hbm, s2f_hbm, d2f_hbm, efw2f_hbm,
             corr_out,
             acc_sh, corr_sh, own_tab,
             ibuf, jbuf, kv, ev, ov,
             map_loc, ownf, idx64,
             srow, drow, accg, tpay, zbuf, zbuf2,
             sem_s, sem_d, sem_a):
    c = lax.axis_index("c")
    s = lax.axis_index("s")
    base = s * EPT

    zvec = jnp.zeros((16,), jnp.float32)
    for cc in range(RD // 16):
        def zb(r, _, cc=cc):
            zbuf[r, pl.ds(cc * 16, 16)] = zvec
            return 0
        lax.fori_loop(0, RPT, zb, 0)
    for cc in range(2 * RD // 16):
        def zb2(r, _, cc=cc):
            zbuf2[r, pl.ds(cc * 16, 16)] = zvec
            return 0
        lax.fori_loop(0, RPT, zb2, 0)

    lane = lax.iota(jnp.int32, 16)
    zivec = jnp.zeros((16,), jnp.int32)

    pltpu.sync_copy(i2d_hbm, ibuf)
    pltpu.sync_copy(j2d_hbm, jbuf)
    for k in range(CH):
        for v in range(CSZ // 16):
            sl = pl.ds(v * 16, 16)
            kv[k, sl] = c * N + ibuf[s * CH + k, sl]
            ev[k, sl] = c * N + jbuf[s * CH + k, sl]
            ov[k, sl] = zivec
    pltpu.sync_copy(ov, own_tab.at[pl.ds(s * CH, CH)])

    def zacc(q, _):
        pltpu.sync_copy(zbuf, acc_sh.at[pl.ds(base + q * RPT, RPT)])
        return 0
    lax.fori_loop(0, EPT // RPT, zacc, 0)
    pltpu.sync_copy(zbuf2, corr_sh.at[pl.ds(s * RPT, RPT)])

    plsc.subcore_barrier()

    def sc1(t, _):
        r = t // 8
        vv = lax.rem(t, 8)
        sl = pl.ds(vv * 16, 16)
        kvec = ibuf[r, sl] * N + jbuf[r, sl]
        m = lax.shift_right_logical(kvec, KSH) == s
        loc = lax.bitwise_and(kvec, KPT - 1)
        plsc.store_scatter(map_loc, [loc], t * 16 + lane, mask=m)
        return 0
    lax.fori_loop(0, E // 16, sc1, 0)

    def sc2(t, _):
        r = t // 8
        vv = lax.rem(t, 8)
        sl = pl.ds(vv * 16, 16)
        kvec = ibuf[r, sl] * N + jbuf[r, sl]
        m = lax.shift_right_logical(kvec, KSH) == s
        loc = lax.bitwise_and(kvec, KPT - 1)
        g = plsc.load_gather(map_loc, [loc], mask=m)
        ownf[r, sl] = jnp.where(m, g, 0)
        return 0
    lax.fori_loop(0, E // 16, sc2, 0)

    for w in range(EC // 16):
        idx64[0, pl.ds(w * 16, 16)] = w * 16 + lane
    pltpu.sync_copy(ownf, own_tab.at[idx64.at[0]], add=True)

    plsc.subcore_barrier()

    pltpu.sync_copy(own_tab.at[pl.ds(s * CH, CH)], ov)
    for k in range(CH):
        off = base + k * CSZ
        pltpu.sync_copy(efw2f_hbm.at[pl.ds(c * E + off, CSZ)], srow)
        pltpu.sync_copy(srow, acc_sh.at[ov.at[k]], add=True)

    plsc.subcore_barrier()

    def zt(r, _):
        for cc in range(RD // 16):
            tpay[r, pl.ds(RD + cc * 16, 16)] = zvec
        return 0
    lax.fori_loop(0, CSZ, zt, 0)

    for k in range(CH):
        off = base + k * CSZ
        ca = pltpu.async_copy(acc_sh.at[pl.ds(off, CSZ)], accg, sem_a)
        cs = pltpu.async_copy(s2f_hbm.at[kv.at[k]], srow, sem_s)
        cd = pltpu.async_copy(d2f_hbm.at[ev.at[k]], drow, sem_d)
        ca.wait()
        cs.wait()
        cd.wait()

        def rb(r, _):
            for cc in range(RD // 16):
                sl = pl.ds(cc * 16, 16)
                b16 = srow[r, sl] + drow[r, sl]
                tpay[r, sl] = (jnp.maximum(b16 + accg[r, sl], 0.0)
                               - jnp.maximum(b16, 0.0))
            return 0
        lax.fori_loop(0, CSZ, rb, 0)
        pltpu.sync_copy(tpay, corr_sh.at[ibuf.at[s * CH + k]], add=True)

    plsc.subcore_barrier()

    pltpu.sync_copy(corr_sh.at[pl.ds(s * RPT, RPT)],
                    corr_out.at[pl.ds(c * N + s * RPT, RPT)])


def _sc_call(i_arr, j_arr, s2, d2, efw2):
    f32 = jnp.float32
    i32 = jnp.int32
    mesh = plsc.VectorSubcoreMesh(core_axis_name="c", subcore_axis_name="s",
                                  num_cores=2, num_subcores=NS)
    fn = pl.kernel(
        _sc_body,
        out_type=jax.ShapeDtypeStruct((2 * N, 2 * RD), f32),
        mesh=mesh,
        compiler_params=pltpu.CompilerParams(use_tc_tiling_on_sc=False, needs_layout_passes=False),
        scratch_types=[
            pltpu.VMEM_SHARED((E, RD), f32),
            pltpu.VMEM_SHARED((N, 2 * RD), f32),
            pltpu.VMEM_SHARED((EC, CSZ), i32),
            pltpu.VMEM((EC, CSZ), i32),
            pltpu.VMEM((EC, CSZ), i32),
            pltpu.VMEM((CH, CSZ), i32),
            pltpu.VMEM((CH, CSZ), i32),
            pltpu.VMEM((CH, CSZ), i32),
            pltpu.VMEM((KPT,), i32),
            pltpu.VMEM((EC, CSZ), i32),
            pltpu.VMEM((1, EC), i32),
            pltpu.VMEM((CSZ, RD), f32),
            pltpu.VMEM((CSZ, RD), f32),
            pltpu.VMEM((CSZ, RD), f32),
            pltpu.VMEM((CSZ, 2 * RD), f32),
            pltpu.VMEM((RPT, RD), f32),
            pltpu.VMEM((RPT, 2 * RD), f32),
            pltpu.SemaphoreType.DMA,
            pltpu.SemaphoreType.DMA,
            pltpu.SemaphoreType.DMA,
        ],
    )
    corr_flat = fn(i_arr.reshape(EC, CSZ), j_arr.reshape(EC, CSZ),
                   s2.reshape(2 * N, RD), d2.reshape(2 * N, RD),
                   efw2.reshape(2 * E, RD))
    return corr_flat.reshape(2, N, 2 * RD)


def _finale_body(s2_ref, d2_ref, corr_ref, hWh2_ref,
                 Pe2_ref, Wgnt2_ref, Wpnt2_ref, Wget2_ref, Wpet2_ref,
                 enW1_ref, enb1_ref, enW2t_ref, enb2_ref,
                 out_ref):
    f32 = jnp.float32
    dot = functools.partial(jnp.dot, preferred_element_type=f32)

    S = jnp.concatenate([s2_ref[0], s2_ref[1]], axis=1)

    IB = 128
    JB = 8
    rows = []
    for ib in range(N // IB):
        Sb = S[ib * IB:(ib + 1) * IB, :]

        def jb_body(jb, acc, Sb=Sb):
            Dblk = jnp.concatenate(
                [d2_ref[0, pl.ds(jb * JB, JB), :],
                 d2_ref[1, pl.ds(jb * JB, JB), :]], axis=1)
            for t in range(JB):
                acc = acc + jnp.maximum(Sb + Dblk[t:t + 1, :], 0.0)
            return acc
        rows.append(lax.fori_loop(0, N // JB, jb_body,
                                  jnp.zeros((IB, 2 * RD), f32)))
    DROW = jnp.concatenate(rows, axis=0)

    pooled = []
    for g in range(2):
        ROW = DROW[:, g * RD:(g + 1) * RD] + corr_ref[g]
        h2 = jnp.maximum(hWh2_ref[g] + dot(ROW, Pe2_ref[g]), 0.0)
        gn = jax.nn.sigmoid(jnp.sum(h2 * Wgnt2_ref[g], axis=1, keepdims=True))
        pn = dot(jnp.sum(gn * h2, axis=0, keepdims=True), Wpnt2_ref[g])
        ge = jax.nn.sigmoid(jnp.sum(ROW * Wget2_ref[g], axis=1, keepdims=True))
        pe = dot(jnp.sum(ge * ROW, axis=0, keepdims=True), Wpet2_ref[g])
        pooled.append(pn + pe)

    hcat = jnp.concatenate(pooled, axis=1)
    e1 = jnp.maximum(dot(hcat, enW1_ref[...]) + enb1_ref[...], 0.0)
    out_ref[...] = dot(e1, enW2t_ref[...]) + enb2_ref[...]


def _finale_call(s2, d2, corr2, hWh2, heads):
    return pl.pallas_call(
        _finale_body,
        out_shape=jax.ShapeDtypeStruct((1, 1), jnp.float32),
    )(s2, d2, corr2, hWh2, *heads)


def kernel(x, union_features, box_info, obj_labels, rel_labels, rel_pair_idx,
           params):
    p = params
    r2 = lambda v: v.reshape(1, -1)

    weights = (
        p['pos_W1'], r2(p['pos_b1']), r2(p['pos_g']), r2(p['pos_bt']),
        p['pos_W2'], r2(p['pos_b2']),
        p['W_obj_emb'], r2(p['b_obj_emb']),
        p['W_obj_lbl'][:NOC], p['W_obj_lbl'][NOC:], r2(p['b_obj_lbl']),
        p['W_rel_emb'], r2(p['b_rel_emb']), p['W_rel_lbl'], r2(p['b_rel_lbl']),
        p['im_Us'], p['im_Ud'], p['im_Wh'], r2(p['im_be']), r2(p['im_bh']), p['im_We'],
        p['sg_Us'], p['sg_Ud'], p['sg_Wh'], r2(p['sg_be']), r2(p['sg_bh']), p['sg_We'],
    )
    s2, d2, hWh2, efW2 = _prep_call(x, union_features, box_info, obj_labels,
                                    rel_labels, weights)

    i_arr = rel_pair_idx[:, 0]
    j_arr = rel_pair_idx[:, 1]
    corr2 = _sc_call(i_arr, j_arr, s2, d2, efW2)[:, :, :RD]

    heads = (
        jnp.stack([p['im_Pe'], p['sg_Pe']]),
        jnp.stack([p['imp_Wgn'].T, p['sgp_Wgn'].T]),
        jnp.stack([p['imp_Wpn'], p['sgp_Wpn']]),
        jnp.stack([p['imp_Wge'].T, p['sgp_Wge'].T]),
        jnp.stack([p['imp_Wpe'], p['sgp_Wpe']]),
        p['en_W1'], r2(p['en_b1']), p['en_W2'], r2(p['en_b2']),
    )
    return _finale_call(s2, d2, corr2, hWh2, heads)
